# Initial kernel scaffold; baseline (speedup 1.0000x reference)
import jax
import jax.numpy as jnp
from jax.experimental import pallas as pl


def kernel(x, tables):
    raise NotImplementedError("write your pallas kernel here")


# speedup vs baseline: 1.1381x; 1.1381x over previous
import functools

import jax
import jax.numpy as jnp
from jax import lax
from jax.experimental import pallas as pl
from jax.experimental.pallas import tpu as pltpu
from jax.experimental.pallas import tpu_sc as plsc

_NUM_FIELDS = 26
_VOCAB = 100000
_EMBED = 32
_BATCH = 16384

_BF = _BATCH * _NUM_FIELDS
_NW = 32
_PER_W = _BF // _NW
_KROW = 8
_CHUNK = _KROW * 128
_NCH = _PER_W // _CHUNK
_NVEC = 128 // 16

_mesh = plsc.VectorSubcoreMesh(core_axis_name="c", subcore_axis_name="s")


@functools.partial(
    pl.kernel,
    mesh=_mesh,
    compiler_params=pltpu.CompilerParams(use_tc_tiling_on_sc=False),
    out_type=jax.ShapeDtypeStruct((_BF, _EMBED), jnp.float32),
    scratch_types=[
        pltpu.VMEM((_KROW, 128), jnp.int32),
        pltpu.VMEM((_PER_W // 128, 128), jnp.int32),
        pltpu.VMEM((_CHUNK, _EMBED), jnp.float32),
        pltpu.SemaphoreType.DMA,
    ],
)
def _emb_kernel(x_hbm, offs_hbm, tab_hbm, out_hbm, idx_v, offs_v, rows_v, sem):
    wid = lax.axis_index("s") * 2 + lax.axis_index("c")
    pltpu.sync_copy(offs_hbm, offs_v)

    def chunk_body(g, carry):
        row0 = wid * (_PER_W // 128) + g * _KROW
        pltpu.sync_copy(x_hbm.at[pl.ds(row0, _KROW)], idx_v)
        for j in range(_KROW):
            for l in range(_NVEC):
                s = pl.ds(l * 16, 16)
                idx_v[j, s] = idx_v[j, s] + offs_v[g * _KROW + j, s]
        copies = []
        for j in range(_KROW):
            copies.append(
                pltpu.async_copy(
                    tab_hbm.at[idx_v.at[j]],
                    rows_v.at[pl.ds(j * 128, 128)],
                    sem,
                )
            )
        for c in copies:
            c.wait()
        out0 = wid * _PER_W + g * _CHUNK
        pltpu.sync_copy(rows_v, out_hbm.at[pl.ds(out0, _CHUNK)])
        return carry

    lax.fori_loop(0, _NCH, chunk_body, 0)


def kernel(x, tables):
    x2 = x.astype(jnp.int32).reshape(_BF // 128, 128)
    tab = tables.reshape(_NUM_FIELDS * _VOCAB, _EMBED)
    offs = (
        jnp.tile(jnp.arange(_NUM_FIELDS, dtype=jnp.int32) * _VOCAB,
                 _PER_W // _NUM_FIELDS)
        .reshape(_PER_W // 128, 128)
    )
    out = _emb_kernel(x2, offs, tab)
    return out.reshape(_BATCH, _NUM_FIELDS, _EMBED)


# speedup vs baseline: 1.7978x; 1.5797x over previous
import functools

import jax
import jax.numpy as jnp
from jax import lax
from jax.experimental import pallas as pl
from jax.experimental.pallas import tpu as pltpu
from jax.experimental.pallas import tpu_sc as plsc

_F = 26
_V = 100000
_E = 32
_B = 16384
_FC = 13
_NB = _FC * 4
_BS = _B // 16

_mesh = plsc.VectorSubcoreMesh(core_axis_name="c", subcore_axis_name="s")


@functools.partial(
    pl.kernel,
    mesh=_mesh,
    compiler_params=pltpu.CompilerParams(use_tc_tiling_on_sc=False),
    out_type=jax.ShapeDtypeStruct((_F * _E, _B), jnp.float32),
    scratch_types=[
        pltpu.VMEM_SHARED((8, _V), jnp.float32),
        pltpu.VMEM((_BS,), jnp.int32),
        pltpu.VMEM((8, _BS), jnp.float32),
        pltpu.SemaphoreType.DMA,
        pltpu.SemaphoreType.DMA,
    ],
)
def _emb_kernel(x_hbm, tab_hbm, out_hbm, sbuf, vidx, obuf, sem_t, sem_g):
    c = lax.axis_index("c")
    s = lax.axis_index("s")
    b0 = s * _BS

    def blk(n, carry):
        f = _FC * c + n // 4
        k = n % 4
        r0 = (f * 4 + k) * 8

        @pl.when(s == 0)
        def _stage():
            pltpu.async_copy(tab_hbm.at[pl.ds(r0, 8), :], sbuf, sem_t).wait()

        @pl.when(k == 0)
        def _load_idx():
            pltpu.sync_copy(x_hbm.at[pl.ds(f * _B + b0, _BS)], vidx)

        plsc.subcore_barrier()
        copies = [
            pltpu.async_copy(sbuf.at[e].at[vidx], obuf.at[e], sem_g)
            for e in range(8)
        ]
        for cp in copies:
            cp.wait()
        pltpu.sync_copy(obuf, out_hbm.at[pl.ds(r0, 8), pl.ds(b0, _BS)])
        plsc.subcore_barrier()
        return carry

    lax.fori_loop(0, _NB, blk, 0)


def kernel(x, tables):
    xt = jnp.swapaxes(x, 0, 1).reshape(_F * _B).astype(jnp.int32)
    tab2 = jnp.swapaxes(tables, 1, 2).reshape(_F * _E, _V)
    out2 = _emb_kernel(xt, tab2)
    return out2.reshape(_F, _E, _B).transpose(2, 0, 1)


# speedup vs baseline: 2.1638x; 1.2036x over previous
import functools

import jax
import jax.numpy as jnp
from jax import lax
from jax.experimental import pallas as pl
from jax.experimental.pallas import tpu as pltpu
from jax.experimental.pallas import tpu_sc as plsc

_F = 26
_V = 100000
_E = 32
_B = 16384
_FC = 13
_NB = _FC * 4
_BS = _B // 16

_mesh = plsc.VectorSubcoreMesh(core_axis_name="c", subcore_axis_name="s")


@functools.partial(
    pl.kernel,
    mesh=_mesh,
    compiler_params=pltpu.CompilerParams(use_tc_tiling_on_sc=False),
    out_type=jax.ShapeDtypeStruct((_F * _E, _B), jnp.float32),
    scratch_types=[
        pltpu.VMEM_SHARED((2, 8, _V), jnp.float32),
        pltpu.VMEM((_BS,), jnp.int32),
        pltpu.VMEM((2, 8, _BS), jnp.float32),
        pltpu.SemaphoreType.DMA,
        pltpu.SemaphoreType.DMA,
        pltpu.SemaphoreType.DMA,
        pltpu.SemaphoreType.DMA,
        pltpu.SemaphoreType.DMA,
    ],
)
def _emb_kernel(x_hbm, tab_hbm, out_hbm, sbuf, vidx, obuf,
                sem_t0, sem_t1, sem_o0, sem_o1, sem_g):
    c = lax.axis_index("c")
    s = lax.axis_index("s")
    b0 = s * _BS
    sem_t = (sem_t0, sem_t1)
    sem_o = (sem_o0, sem_o1)

    def stage(n, buf):
        return pltpu.async_copy(
            tab_hbm.at[pl.ds((_NB * c + n) * 8, 8), :], sbuf.at[buf], sem_t[buf]
        )

    @pl.when(s == 0)
    def _prologue():
        stage(0, 0)

    def pair(g, carry):
        for b in (0, 1):
            n = 2 * g + b

            @pl.when(s == 0)
            def _wait_stage():
                pltpu.make_async_copy(
                    tab_hbm.at[pl.ds(0, 8), :], sbuf.at[b], sem_t[b]
                ).wait()

            plsc.subcore_barrier()

            @pl.when((s == 0) & (n + 1 < _NB))
            def _stage_next():
                stage(n + 1, 1 - b)

            if b == 0:
                @pl.when(g % 2 == 0)
                def _load_idx():
                    f = _FC * c + n // 4
                    pltpu.sync_copy(x_hbm.at[pl.ds(f * _B + b0, _BS)], vidx)

            @pl.when(n >= 2)
            def _wait_out():
                pltpu.make_async_copy(
                    obuf.at[b], out_hbm.at[pl.ds(0, 8), pl.ds(0, _BS)], sem_o[b]
                ).wait()

            copies = [
                pltpu.async_copy(
                    sbuf.at[b].at[e].at[vidx], obuf.at[b, e], sem_g
                )
                for e in range(8)
            ]
            for cp in copies:
                cp.wait()
            r0 = (_NB * c + n) * 8
            pltpu.async_copy(
                obuf.at[b], out_hbm.at[pl.ds(r0, 8), pl.ds(b0, _BS)], sem_o[b]
            )
            plsc.subcore_barrier()
        return carry

    lax.fori_loop(0, _NB // 2, pair, 0)
    pltpu.make_async_copy(
        obuf.at[0], out_hbm.at[pl.ds(0, 8), pl.ds(0, _BS)], sem_o[0]
    ).wait()
    pltpu.make_async_copy(
        obuf.at[1], out_hbm.at[pl.ds(0, 8), pl.ds(0, _BS)], sem_o[1]
    ).wait()


def kernel(x, tables):
    xt = jnp.swapaxes(x, 0, 1).reshape(_F * _B).astype(jnp.int32)
    tab2 = jnp.swapaxes(tables, 1, 2).reshape(_F * _E, _V)
    out2 = _emb_kernel(xt, tab2)
    return out2.reshape(_F, _E, _B).transpose(2, 0, 1)
